# baseline (speedup 1.0000x reference)
import functools

import jax
import jax.numpy as jnp
from jax import lax
from jax.experimental import pallas as pl
from jax.experimental.pallas import tpu as pltpu
from jax.experimental.pallas import tpu_sc as plsc

_N = 10000
_D_IN = 128
_D_EDGE = 16
_D_M = 128
_D_OUT = 256
_EPS = 1e-5

_NC = 2
_NS = 16
_L = 16
_G = 128
_DH = _D_M // _NC

_ROWS_PER_TILE = ((_N + 1 + _NS - 1) // _NS + 7) // 8 * 8
_R_PAD = _ROWS_PER_TILE * _NS
_TRASH = _N


def _sc_edge_kernel(xwh, ew, src2, dst2, zrow, groups_per_tile, e_pad):
  chunks = groups_per_tile
  mesh = plsc.VectorSubcoreMesh(core_axis_name="c", subcore_axis_name="s")

  @functools.partial(
      pl.kernel,
      mesh=mesh,
      compiler_params=pltpu.CompilerParams(use_tc_tiling_on_sc=False),
      out_type=jax.ShapeDtypeStruct((_NC * _R_PAD, _DH), jnp.float32),
      scratch_types=[
          pltpu.VMEM((groups_per_tile, _G), jnp.int32),
          pltpu.VMEM((2, _G), jnp.int32),
          pltpu.VMEM((2, _G, _DH), jnp.float32),
          pltpu.VMEM((2, _G, _D_M), jnp.float32),
          pltpu.VMEM((_G, _DH), jnp.float32),
          pltpu.VMEM_SHARED((_R_PAD, _DH), jnp.float32),
          pltpu.SemaphoreType.DMA,
          pltpu.SemaphoreType.DMA,
      ],
  )
  def body(xw_hbm, ew_hbm, src_hbm, dst_hbm, z_hbm, out_hbm,
           sidx, didxb, rows, ews, msgs, agg, sem0, sem1):
    c = lax.axis_index("c")
    s = lax.axis_index("s")

    row_off = pl.multiple_of(s * _ROWS_PER_TILE, 8)
    pltpu.sync_copy(z_hbm, agg.at[pl.ds(row_off, _ROWS_PER_TILE)])
    g0 = pl.multiple_of(s * groups_per_tile, 8)
    pltpu.sync_copy(src_hbm.at[pl.ds(g0, groups_per_tile)], sidx)
    base = c * _N

    @plsc.parallel_loop(0, groups_per_tile, 1, unroll=4)
    def _(r):
      for cc in range(_G // _L):
        sl = pl.ds(cc * _L, _L)
        sidx[r, sl] = sidx[r, sl] + base

    plsc.subcore_barrier()

    ebase = pl.multiple_of(s * groups_per_tile * _G, 8)

    hoff = pl.multiple_of(c * _DH, 8)

    def start(g, b, sem):
      eoff = pl.multiple_of(ebase + g * _G, 8)
      pltpu.async_copy(ew_hbm.at[pl.ds(eoff, _G)], ews.at[b], sem)
      pltpu.async_copy(dst_hbm.at[pl.ds(eoff, _G)], didxb.at[b], sem)
      pltpu.async_copy(xw_hbm.at[sidx.at[g]], rows.at[b], sem)

    def wait(b, sem):
      pltpu.make_async_copy(ew_hbm.at[pl.ds(0, _G)], ews.at[b], sem).wait()
      pltpu.make_async_copy(dst_hbm.at[pl.ds(0, _G)], didxb.at[b], sem).wait()
      pltpu.make_async_copy(xw_hbm.at[pl.ds(0, _G)], rows.at[b], sem).wait()

    def compute(b):
      @plsc.parallel_loop(0, _G, 1, unroll=8)
      def _(r):
        for cc in range(_DH // _L):
          sl = pl.ds(cc * _L, _L)
          esl = pl.ds(hoff + cc * _L, _L)
          msgs[r, sl] = jnp.maximum(rows[b, r, sl] + ews[b, r, esl], 0.0)

    def scatter(g, b):
      pltpu.sync_copy(msgs, agg.at[didxb.at[b]], add=True)

    start(0, 0, sem0)

    def pair(h, carry):
      ga = 2 * h
      gb = 2 * h + 1
      start(gb, 1, sem1)
      wait(0, sem0)
      compute(0)
      scatter(ga, 0)

      @pl.when(gb + 1 < chunks)
      def _():
        start(gb + 1, 0, sem0)

      wait(1, sem1)
      compute(1)
      scatter(gb, 1)
      return carry

    lax.fori_loop(0, chunks // 2, pair, 0)
    plsc.subcore_barrier()

    off = pl.multiple_of(c * _R_PAD + s * _ROWS_PER_TILE, 8)
    pltpu.sync_copy(agg.at[pl.ds(row_off, _ROWS_PER_TILE)],
                    out_hbm.at[pl.ds(off, _ROWS_PER_TILE)])

  return body(xwh, ew, src2, dst2, zrow)


def _xw_body(x_ref, w_ref, o_ref):
  xw = jnp.dot(x_ref[...], w_ref[...], preferred_element_type=jnp.float32)
  o_ref[0:_N, :] = xw[:, :_DH]
  o_ref[_N:, :] = xw[:, _DH:]


def _ew_body(ef_ref, w_ref, b_ref, o_ref):
  o_ref[...] = jnp.dot(ef_ref[...], w_ref[...],
                       preferred_element_type=jnp.float32) + b_ref[...]


def _pass1_body(x_ref, a0_ref, a1_ref, w_ref, b_ref, pre_ref, st_ref):
  i = pl.program_id(0)
  z = jnp.concatenate([x_ref[...], a0_ref[0], a1_ref[0]], axis=1)
  pre = jnp.dot(z, w_ref[...], preferred_element_type=jnp.float32)
  pre = pre + b_ref[...] + z
  pre_ref[...] = pre

  @pl.when(i == 0)
  def _():
    st_ref[...] = jnp.zeros_like(st_ref)

  s1 = jnp.sum(pre, axis=0, keepdims=True)
  s2 = jnp.sum(pre * pre, axis=0, keepdims=True)
  st_ref[...] += jnp.concatenate([s1, s2], axis=0)


def _pass2_body(pre_ref, st_ref, g_ref, b_ref, o_ref):
  mean = st_ref[0:1, :] * (1.0 / _N)
  var = st_ref[1:2, :] * (1.0 / _N) - mean * mean
  inv = lax.rsqrt(var + _EPS)
  o_ref[...] = (pre_ref[...] - mean) * (inv * g_ref[...]) + b_ref[...]


def kernel(x, edge_features, edge_idx, batch_idx, W_M, b_M, W_U, b_U,
           gamma, beta):
  del batch_idx
  n = x.shape[0]
  e = edge_features.shape[0]
  assert n == _N

  e_quant = 8 * _NS * _G
  e_pad = ((e + e_quant - 1) // e_quant) * e_quant
  groups_per_tile = e_pad // (_NS * _G)
  src = jnp.pad(edge_idx[0], (0, e_pad - e))
  dst = jnp.pad(edge_idx[1], (0, e_pad - e), constant_values=_TRASH)
  src2 = src.reshape(e_pad // _G, _G)
  w_mx_t = W_M[:, :_D_IN].T
  w_me_t = W_M[:, _D_IN:].T
  b_m = b_M.reshape(1, _D_M)
  w_u_t = W_U.T
  b_u = b_U.reshape(1, _D_OUT)
  gamma2 = gamma.reshape(1, _D_OUT)
  beta2 = beta.reshape(1, _D_OUT)
  zrow = jnp.zeros((_ROWS_PER_TILE, _DH), jnp.float32)

  xwh2 = pl.pallas_call(
      _xw_body,
      out_shape=jax.ShapeDtypeStruct((_NC * n, _DH), jnp.float32),
  )(x, w_mx_t)

  be = 4096
  ge = (e + be - 1) // be
  ew = pl.pallas_call(
      _ew_body,
      grid=(ge,),
      in_specs=[
          pl.BlockSpec((be, _D_EDGE), lambda i: (i, 0)),
          pl.BlockSpec((_D_EDGE, _D_M), lambda i: (0, 0)),
          pl.BlockSpec((1, _D_M), lambda i: (0, 0)),
      ],
      out_specs=pl.BlockSpec((be, _D_M), lambda i: (i, 0)),
      out_shape=jax.ShapeDtypeStruct((e_pad, _D_M), jnp.float32),
  )(edge_features, w_me_t, b_m)

  parts_flat = _sc_edge_kernel(xwh2, ew, src2, dst, zrow,
                               groups_per_tile, e_pad)
  parts = parts_flat.reshape(_NC, _R_PAD, _DH)

  bn = 1000
  gn = n // bn
  pre, stats = pl.pallas_call(
      _pass1_body,
      grid=(gn,),
      in_specs=[
          pl.BlockSpec((bn, _D_IN), lambda i: (i, 0)),
          pl.BlockSpec((1, bn, _DH), lambda i: (0, i, 0)),
          pl.BlockSpec((1, bn, _DH), lambda i: (1, i, 0)),
          pl.BlockSpec((_D_OUT, _D_OUT), lambda i: (0, 0)),
          pl.BlockSpec((1, _D_OUT), lambda i: (0, 0)),
      ],
      out_specs=[
          pl.BlockSpec((bn, _D_OUT), lambda i: (i, 0)),
          pl.BlockSpec((2, _D_OUT), lambda i: (0, 0)),
      ],
      out_shape=[
          jax.ShapeDtypeStruct((n, _D_OUT), jnp.float32),
          jax.ShapeDtypeStruct((2, _D_OUT), jnp.float32),
      ],
  )(x, parts, parts, w_u_t, b_u)

  out = pl.pallas_call(
      _pass2_body,
      grid=(gn,),
      in_specs=[
          pl.BlockSpec((bn, _D_OUT), lambda i: (i, 0)),
          pl.BlockSpec((2, _D_OUT), lambda i: (0, 0)),
          pl.BlockSpec((1, _D_OUT), lambda i: (0, 0)),
          pl.BlockSpec((1, _D_OUT), lambda i: (0, 0)),
      ],
      out_specs=pl.BlockSpec((bn, _D_OUT), lambda i: (i, 0)),
      out_shape=jax.ShapeDtypeStruct((n, _D_OUT), jnp.float32),
  )(pre, stats, gamma2, beta2)
  return out


# speedup vs baseline: 1.0732x; 1.0732x over previous
import functools

import jax
import jax.numpy as jnp
from jax import lax
from jax.experimental import pallas as pl
from jax.experimental.pallas import tpu as pltpu
from jax.experimental.pallas import tpu_sc as plsc

_N = 10000
_D_IN = 128
_D_EDGE = 16
_D_M = 128
_D_OUT = 256
_EPS = 1e-5

_NC = 2
_NS = 16
_L = 16
_G = 128
_DH = _D_M // _NC

_ROWS_PER_TILE = ((_N + 1 + _NS - 1) // _NS + 7) // 8 * 8
_R_PAD = _ROWS_PER_TILE * _NS
_TRASH = _N


def _sc_edge_kernel(xwh, ew0, ew1, src2, dst1, zrow, groups_per_tile, e_pad):
  chunks = groups_per_tile
  mesh = plsc.VectorSubcoreMesh(core_axis_name="c", subcore_axis_name="s")

  @functools.partial(
      pl.kernel,
      mesh=mesh,
      compiler_params=pltpu.CompilerParams(use_tc_tiling_on_sc=False),
      out_type=jax.ShapeDtypeStruct((_NC * _R_PAD, _DH), jnp.float32),
      scratch_types=[
          pltpu.VMEM((groups_per_tile, _G), jnp.int32),
          pltpu.VMEM((2, _G), jnp.int32),
          pltpu.VMEM((2, _G, _DH), jnp.float32),
          pltpu.VMEM((2, _G // 2, _D_M), jnp.float32),
          pltpu.VMEM((_G, _DH), jnp.float32),
          pltpu.VMEM_SHARED((_R_PAD, _DH), jnp.float32),
          pltpu.SemaphoreType.DMA,
          pltpu.SemaphoreType.DMA,
      ],
  )
  def body(xw_hbm, ew0_hbm, ew1_hbm, src_hbm, dst_hbm, z_hbm, out_hbm,
           sidx, didxb, rows, ews, msgs, agg, sem0, sem1):
    c = lax.axis_index("c")
    s = lax.axis_index("s")

    row_off = pl.multiple_of(s * _ROWS_PER_TILE, 8)
    pltpu.sync_copy(z_hbm, agg.at[pl.ds(row_off, _ROWS_PER_TILE)])
    g0 = pl.multiple_of(s * groups_per_tile, 8)
    pltpu.sync_copy(src_hbm.at[pl.ds(g0, groups_per_tile)], sidx)
    base = c * _N

    @plsc.parallel_loop(0, groups_per_tile, 1, unroll=4)
    def _(r):
      for cc in range(_G // _L):
        sl = pl.ds(cc * _L, _L)
        sidx[r, sl] = sidx[r, sl] + base

    plsc.subcore_barrier()

    ebase = pl.multiple_of(s * groups_per_tile * _G, 8)

    def start(g, b, sem):
      eoff = pl.multiple_of(ebase + g * _G, 8)
      eoff2 = pl.multiple_of((ebase + g * _G) // 2, 8)

      @pl.when(c == 0)
      def _():
        pltpu.async_copy(ew0_hbm.at[pl.ds(eoff2, _G // 2)], ews.at[b], sem)

      @pl.when(c == 1)
      def _():
        pltpu.async_copy(ew1_hbm.at[pl.ds(eoff2, _G // 2)], ews.at[b], sem)

      pltpu.async_copy(dst_hbm.at[pl.ds(eoff, _G)], didxb.at[b], sem)
      pltpu.async_copy(xw_hbm.at[sidx.at[g]], rows.at[b], sem)

    def wait(b, sem):
      pltpu.make_async_copy(ew0_hbm.at[pl.ds(0, _G // 2)], ews.at[b],
                            sem).wait()
      pltpu.make_async_copy(dst_hbm.at[pl.ds(0, _G)], didxb.at[b], sem).wait()
      pltpu.make_async_copy(xw_hbm.at[pl.ds(0, _G)], rows.at[b], sem).wait()

    def compute(b):
      @plsc.parallel_loop(0, _G // 2, 1, unroll=4)
      def _(r):
        for half in range(2):
          for cc in range(_DH // _L):
            sl = pl.ds(cc * _L, _L)
            esl = pl.ds(half * _DH + cc * _L, _L)
            eidx = half * (_G // 2) + r
            msgs[eidx, sl] = jnp.maximum(rows[b, eidx, sl] + ews[b, r, esl],
                                         0.0)

    def scatter(g, b):
      pltpu.sync_copy(msgs, agg.at[didxb.at[b]], add=True)

    start(0, 0, sem0)

    def pair(h, carry):
      ga = 2 * h
      gb = 2 * h + 1
      start(gb, 1, sem1)
      wait(0, sem0)
      compute(0)
      scatter(ga, 0)

      @pl.when(gb + 1 < chunks)
      def _():
        start(gb + 1, 0, sem0)

      wait(1, sem1)
      compute(1)
      scatter(gb, 1)
      return carry

    lax.fori_loop(0, chunks // 2, pair, 0)
    plsc.subcore_barrier()

    off = pl.multiple_of(c * _R_PAD + s * _ROWS_PER_TILE, 8)
    pltpu.sync_copy(agg.at[pl.ds(row_off, _ROWS_PER_TILE)],
                    out_hbm.at[pl.ds(off, _ROWS_PER_TILE)])

  return body(xwh, ew0, ew1, src2, dst1, zrow)


def _xw_body(x_ref, w_ref, o_ref):
  xw = jnp.dot(x_ref[...], w_ref[...], preferred_element_type=jnp.float32)
  o_ref[0:_N, :] = xw[:, :_DH]
  o_ref[_N:, :] = xw[:, _DH:]


def _ew_body(ef_ref, w_ref, b_ref, o0_ref, o1_ref):
  ew = jnp.dot(ef_ref[...], w_ref[...],
               preferred_element_type=jnp.float32) + b_ref[...]
  be = ef_ref.shape[0]
  for k in range(be // _G):
    a = ew[k * _G:k * _G + _G // 2]
    b = ew[k * _G + _G // 2:(k + 1) * _G]
    h = _G // 2
    o0_ref[k * h:(k + 1) * h, :] = jnp.concatenate(
        [a[:, :_DH], b[:, :_DH]], axis=1)
    o1_ref[k * h:(k + 1) * h, :] = jnp.concatenate(
        [a[:, _DH:], b[:, _DH:]], axis=1)


def _pass1_body(x_ref, a0_ref, a1_ref, w_ref, b_ref, pre_ref, st_ref):
  i = pl.program_id(0)
  z = jnp.concatenate([x_ref[...], a0_ref[0], a1_ref[0]], axis=1)
  pre = jnp.dot(z, w_ref[...], preferred_element_type=jnp.float32)
  pre = pre + b_ref[...] + z
  pre_ref[...] = pre

  @pl.when(i == 0)
  def _():
    st_ref[...] = jnp.zeros_like(st_ref)

  s1 = jnp.sum(pre, axis=0, keepdims=True)
  s2 = jnp.sum(pre * pre, axis=0, keepdims=True)
  st_ref[...] += jnp.concatenate([s1, s2], axis=0)


def _pass2_body(pre_ref, st_ref, g_ref, b_ref, o_ref):
  mean = st_ref[0:1, :] * (1.0 / _N)
  var = st_ref[1:2, :] * (1.0 / _N) - mean * mean
  inv = lax.rsqrt(var + _EPS)
  o_ref[...] = (pre_ref[...] - mean) * (inv * g_ref[...]) + b_ref[...]


def kernel(x, edge_features, edge_idx, batch_idx, W_M, b_M, W_U, b_U,
           gamma, beta):
  del batch_idx
  n = x.shape[0]
  e = edge_features.shape[0]
  assert n == _N

  e_quant = 8 * _NS * _G
  e_pad = ((e + e_quant - 1) // e_quant) * e_quant
  groups_per_tile = e_pad // (_NS * _G)
  src = jnp.pad(edge_idx[0], (0, e_pad - e))
  dst = jnp.pad(edge_idx[1], (0, e_pad - e), constant_values=_TRASH)
  src2 = src.reshape(e_pad // _G, _G)
  w_mx_t = W_M[:, :_D_IN].T
  w_me_t = W_M[:, _D_IN:].T
  b_m = b_M.reshape(1, _D_M)
  w_u_t = W_U.T
  b_u = b_U.reshape(1, _D_OUT)
  gamma2 = gamma.reshape(1, _D_OUT)
  beta2 = beta.reshape(1, _D_OUT)
  zrow = jnp.zeros((_ROWS_PER_TILE, _DH), jnp.float32)

  xwh2 = pl.pallas_call(
      _xw_body,
      out_shape=jax.ShapeDtypeStruct((_NC * n, _DH), jnp.float32),
  )(x, w_mx_t)

  be = 4096
  ge = (e + be - 1) // be
  ew0, ew1 = pl.pallas_call(
      _ew_body,
      grid=(ge,),
      in_specs=[
          pl.BlockSpec((be, _D_EDGE), lambda i: (i, 0)),
          pl.BlockSpec((_D_EDGE, _D_M), lambda i: (0, 0)),
          pl.BlockSpec((1, _D_M), lambda i: (0, 0)),
      ],
      out_specs=[
          pl.BlockSpec((be // 2, _D_M), lambda i: (i, 0)),
          pl.BlockSpec((be // 2, _D_M), lambda i: (i, 0)),
      ],
      out_shape=[
          jax.ShapeDtypeStruct((e_pad // 2, _D_M), jnp.float32),
          jax.ShapeDtypeStruct((e_pad // 2, _D_M), jnp.float32),
      ],
  )(edge_features, w_me_t, b_m)

  parts_flat = _sc_edge_kernel(xwh2, ew0, ew1, src2, dst, zrow,
                               groups_per_tile, e_pad)
  parts = parts_flat.reshape(_NC, _R_PAD, _DH)

  bn = 1000
  gn = n // bn
  pre, stats = pl.pallas_call(
      _pass1_body,
      grid=(gn,),
      in_specs=[
          pl.BlockSpec((bn, _D_IN), lambda i: (i, 0)),
          pl.BlockSpec((1, bn, _DH), lambda i: (0, i, 0)),
          pl.BlockSpec((1, bn, _DH), lambda i: (1, i, 0)),
          pl.BlockSpec((_D_OUT, _D_OUT), lambda i: (0, 0)),
          pl.BlockSpec((1, _D_OUT), lambda i: (0, 0)),
      ],
      out_specs=[
          pl.BlockSpec((bn, _D_OUT), lambda i: (i, 0)),
          pl.BlockSpec((2, _D_OUT), lambda i: (0, 0)),
      ],
      out_shape=[
          jax.ShapeDtypeStruct((n, _D_OUT), jnp.float32),
          jax.ShapeDtypeStruct((2, _D_OUT), jnp.float32),
      ],
  )(x, parts, parts, w_u_t, b_u)

  out = pl.pallas_call(
      _pass2_body,
      grid=(gn,),
      in_specs=[
          pl.BlockSpec((bn, _D_OUT), lambda i: (i, 0)),
          pl.BlockSpec((2, _D_OUT), lambda i: (0, 0)),
          pl.BlockSpec((1, _D_OUT), lambda i: (0, 0)),
          pl.BlockSpec((1, _D_OUT), lambda i: (0, 0)),
      ],
      out_specs=pl.BlockSpec((bn, _D_OUT), lambda i: (i, 0)),
      out_shape=jax.ShapeDtypeStruct((n, _D_OUT), jnp.float32),
  )(pre, stats, gamma2, beta2)
  return out
